# Initial kernel scaffold; baseline (speedup 1.0000x reference)
import jax
import jax.numpy as jnp
from jax.experimental import pallas as pl


def kernel(target_item, target_cate, target_time, hist_item, hist_cate, hist_time, mask, item_table, cate_table, time_table):
    raise NotImplementedError("write your pallas kernel here")


# speedup vs baseline: 3.3845x; 3.3845x over previous
import functools
import math

import jax
import jax.numpy as jnp
from jax import lax
from jax.experimental import pallas as pl
from jax.experimental.pallas import tpu as pltpu
from jax.experimental.pallas import tpu_sc as plsc

B = 4096
L = 200
D = 16

_NC = 2
_NS = 16
_NW = _NC * _NS

_HIST = B * L
_HPW = _HIST // _NW
_CH = 2560
_NSTEP = _HPW // _CH
_G = 128
_NG = _CH // _G
_TPW = B // _NW


def _gather_body(hist_item, hist_cate, hist_time, t_item, t_cate, t_time,
                 item_tb, cate_tb, time_tb,
                 hi_rows, hc_rows, ht_rows, ti_rows, tc_rows, tt_rows,
                 idx_i, idx_c, idx_t, rows_i, rows_c, rows_t, sem):
    wid = lax.axis_index("s") * _NC + lax.axis_index("c")
    hbase = wid * _HPW

    hists = (hist_item, hist_cate, hist_time)
    tables = (item_tb, cate_tb, time_tb)
    houts = (hi_rows, hc_rows, ht_rows)
    idxs = (idx_i, idx_c, idx_t)
    rows = (rows_i, rows_c, rows_t)

    def step(i, carry):
        off = hbase + i * _CH
        for t in range(3):
            pltpu.sync_copy(hists[t].at[pl.ds(off, _CH)], idxs[t])
            copies = []
            for j in range(_NG):
                copies.append(pltpu.async_copy(
                    tables[t].at[idxs[t].at[pl.ds(j * _G, _G)]],
                    rows[t].at[pl.ds(j * _G, _G)], sem))
            for c in copies:
                c.wait()
            pltpu.sync_copy(rows[t], houts[t].at[pl.ds(off, _CH)])
        return carry

    lax.fori_loop(0, _NSTEP, step, 0)

    tbase = wid * _TPW
    tins = (t_item, t_cate, t_time)
    touts = (ti_rows, tc_rows, tt_rows)
    for t in range(3):
        pltpu.sync_copy(tins[t].at[pl.ds(tbase, _TPW)],
                        idxs[t].at[pl.ds(0, _TPW)])
        pltpu.async_copy(tables[t].at[idxs[t].at[pl.ds(0, _TPW)]],
                         rows[t].at[pl.ds(0, _TPW)], sem).wait()
        pltpu.sync_copy(rows[t].at[pl.ds(0, _TPW)],
                        touts[t].at[pl.ds(tbase, _TPW)])


def _sc_gather(hist_item, hist_cate, hist_time, t_item, t_cate, t_time,
               item_tb, cate_tb, time_tb):
    mesh = plsc.VectorSubcoreMesh(core_axis_name="c", subcore_axis_name="s")
    f32 = jnp.float32
    out_type = (
        jax.ShapeDtypeStruct((_HIST, D), f32),
        jax.ShapeDtypeStruct((_HIST, D), f32),
        jax.ShapeDtypeStruct((_HIST, D), f32),
        jax.ShapeDtypeStruct((B, D), f32),
        jax.ShapeDtypeStruct((B, D), f32),
        jax.ShapeDtypeStruct((B, D), f32),
    )
    scratch = [
        pltpu.VMEM((_CH,), jnp.int32),
        pltpu.VMEM((_CH,), jnp.int32),
        pltpu.VMEM((_CH,), jnp.int32),
        pltpu.VMEM((_CH, D), f32),
        pltpu.VMEM((_CH, D), f32),
        pltpu.VMEM((_CH, D), f32),
        pltpu.SemaphoreType.DMA,
    ]
    fn = pl.kernel(_gather_body, mesh=mesh, out_type=out_type,
                   scratch_types=scratch,
                   compiler_params=pltpu.CompilerParams(
                       use_tc_tiling_on_sc=False))
    return fn(hist_item, hist_cate, hist_time, t_item, t_cate, t_time,
              item_tb, cate_tb, time_tb)


_BB = 64


def _attn_body(hi_ref, hc_ref, ht_ref, ti_ref, tc_ref, tt_ref, mask_ref,
               out_ref):
    hi = hi_ref[...]
    hc = hc_ref[...]
    ht = ht_ref[...]
    ti = ti_ref[...]
    tc = tc_ref[...]
    tt = tt_ref[...]
    scale = 1.0 / math.sqrt(3 * D)
    scores = (jnp.sum(hi * ti[:, None, :], axis=-1)
              + jnp.sum(hc * tc[:, None, :], axis=-1)
              + jnp.sum(ht * tt[:, None, :], axis=-1)) * scale
    scores = jnp.where(mask_ref[...] > 0, scores, jnp.float32(-1e9))
    m = jnp.max(scores, axis=-1, keepdims=True)
    e = jnp.exp(scores - m)
    attn = e / jnp.sum(e, axis=-1, keepdims=True)
    pi = jnp.sum(attn[:, :, None] * hi, axis=1)
    pc = jnp.sum(attn[:, :, None] * hc, axis=1)
    pt = jnp.sum(attn[:, :, None] * ht, axis=1)
    out_ref[...] = jnp.concatenate([ti, tc, tt, pi, pc, pt], axis=-1)


def _tc_attention(hi, hc, ht, ti, tc, tt, mask):
    grid = (B // _BB,)
    hist_spec = pl.BlockSpec((_BB, L, D), lambda i: (i, 0, 0))
    tgt_spec = pl.BlockSpec((_BB, D), lambda i: (i, 0))
    mask_spec = pl.BlockSpec((_BB, L), lambda i: (i, 0))
    out_spec = pl.BlockSpec((_BB, 6 * D), lambda i: (i, 0))
    return pl.pallas_call(
        _attn_body,
        grid=grid,
        in_specs=[hist_spec, hist_spec, hist_spec,
                  tgt_spec, tgt_spec, tgt_spec, mask_spec],
        out_specs=out_spec,
        out_shape=jax.ShapeDtypeStruct((B, 6 * D), jnp.float32),
    )(hi, hc, ht, ti, tc, tt, mask)


def kernel(target_item, target_cate, target_time, hist_item, hist_cate,
           hist_time, mask, item_table, cate_table, time_table):
    hi, hc, ht, ti, tc, tt = _sc_gather(
        hist_item.reshape(-1), hist_cate.reshape(-1), hist_time.reshape(-1),
        target_item, target_cate, target_time,
        item_table, cate_table, time_table)
    hi = hi.reshape(B, L, D)
    hc = hc.reshape(B, L, D)
    ht = ht.reshape(B, L, D)
    return _tc_attention(hi, hc, ht, ti, tc, tt, mask)


# speedup vs baseline: 8.4202x; 2.4879x over previous
import math

import jax
import jax.numpy as jnp
from jax import lax
from jax.experimental import pallas as pl
from jax.experimental.pallas import tpu as pltpu
from jax.experimental.pallas import tpu_sc as plsc

B = 4096
L = 200
D = 16
LP = 208

_NC = 2
_NS = 16
_NW = _NC * _NS
_RPW = B // _NW
_G = 4
_NSS = _RPW // _G
_SSL = _G * LP
_NCH = 13
_GCH = [(o, min(128, _SSL - o)) for o in range(0, _SSL, 128)]

_SCALE = 1.0 / math.sqrt(3 * D)
_NEG = -1e9


def _fused_body(hist_item, hist_cate, hist_time, t_item, t_cate, t_time,
                mask_flat, item_tb, cate_tb, time_tb, out_hbm,
                idx_bufs, mask_bufs, rows_bufs, t_rows, tidx_v,
                attn_v, out_v, sem_t, sems_idx, sems_g):
    wid = lax.axis_index("s") * _NC + lax.axis_index("c")
    row0 = wid * _RPW
    hists = (hist_item, hist_cate, hist_time)
    tables = (item_tb, cate_tb, time_tb)

    zero16 = jnp.zeros((16,), jnp.float32)
    zero16i = jnp.zeros((16,), jnp.int32)

    def zloop(i, c):
        for p in range(2):
            for t in range(3):
                idx_bufs[p][t][pl.ds(i * 16, 16)] = zero16i
            mask_bufs[p][pl.ds(i * 16, 16)] = zero16
        return c
    lax.fori_loop(0, _SSL // 16, zloop, 0)

    tgts = (t_item, t_cate, t_time)
    for t in range(3):
        pltpu.sync_copy(tgts[t].at[pl.ds(row0, _RPW)], tidx_v)
        pltpu.async_copy(tables[t].at[tidx_v], t_rows[t], sem_t).wait()

    def idx_copies(s, p):
        cps = []
        for r in range(_G):
            src_off = (row0 + s * _G + r) * L
            dst_off = r * LP
            for t in range(3):
                cps.append(pltpu.make_async_copy(
                    hists[t].at[pl.ds(src_off, L)],
                    idx_bufs[p][t].at[pl.ds(dst_off, L)], sems_idx[p]))
            cps.append(pltpu.make_async_copy(
                mask_flat.at[pl.ds(src_off, L)],
                mask_bufs[p].at[pl.ds(dst_off, L)], sems_idx[p]))
        return cps

    def gath_copies(p):
        cps = []
        for t in range(3):
            for (o, n) in _GCH:
                cps.append(pltpu.make_async_copy(
                    tables[t].at[idx_bufs[p][t].at[pl.ds(o, n)]],
                    rows_bufs[p][t].at[pl.ds(o, n)], sems_g[p]))
        return cps

    def start(cps):
        for c in cps:
            c.start()

    def wait(cps):
        for c in cps:
            c.wait()

    start(idx_copies(0, 0))
    start(idx_copies(1, 1))
    wait(idx_copies(0, 0))
    start(gath_copies(0))

    def compute_row(s, p, r4):
        base = r4 * LP
        rr = s * _G + r4
        tiu = t_rows[0][rr]
        tcu = t_rows[1][rr]
        ttu = t_rows[2][rr]
        ti = tiu * _SCALE
        tc = tcu * _SCALE
        tt = ttu * _SCALE
        ri_ref = rows_bufs[p][0]
        rc_ref = rows_bufs[p][1]
        rt_ref = rows_bufs[p][2]

        lanes = lax.iota(jnp.int32, 16)

        def sloop(c, carry):
            lb = base + c * 16
            vec = jnp.zeros((16,), jnp.float32)
            for j in range(16):
                sj = jnp.sum(ri_ref[lb + j] * ti + rc_ref[lb + j] * tc
                             + rt_ref[lb + j] * tt)
                vec = jnp.where(lanes == j, sj, vec)
            attn_v[pl.ds(c * 16, 16)] = vec
            return carry
        lax.fori_loop(0, _NCH, sloop, 0)

        macc = jnp.full((16,), _NEG, jnp.float32)
        for c in range(_NCH):
            sc = attn_v[pl.ds(c * 16, 16)]
            mv = mask_bufs[p][pl.ds(base + c * 16, 16)]
            macc = jnp.maximum(macc, jnp.where(mv > 0, sc, _NEG))
        m = jnp.max(macc)

        sacc = jnp.zeros((16,), jnp.float32)
        for c in range(_NCH):
            sc = attn_v[pl.ds(c * 16, 16)]
            mv = mask_bufs[p][pl.ds(base + c * 16, 16)]
            ec = jnp.exp(jnp.where(mv > 0, sc, _NEG) - m)
            attn_v[pl.ds(c * 16, 16)] = ec
            sacc = sacc + ec
        denom = jnp.sum(sacc)
        inv = jnp.ones((16,), jnp.float32) / denom

        def ploop(c, carry):
            ai, ac, at = carry
            lb = c * 16
            av = attn_v[pl.ds(lb, 16)]
            for j in range(16):
                a = av[j]
                ai = ai + a * ri_ref[base + lb + j]
                ac = ac + a * rc_ref[base + lb + j]
                at = at + a * rt_ref[base + lb + j]
            return (ai, ac, at)
        z = jnp.zeros((16,), jnp.float32)
        ai, ac, at = lax.fori_loop(0, _NCH, ploop, (z, z, z))

        out_v[rr, pl.ds(0, 16)] = tiu
        out_v[rr, pl.ds(16, 16)] = tcu
        out_v[rr, pl.ds(32, 16)] = ttu
        out_v[rr, pl.ds(48, 16)] = ai * inv
        out_v[rr, pl.ds(64, 16)] = ac * inv
        out_v[rr, pl.ds(80, 16)] = at * inv

    def body(i, carry):
        for p in range(2):
            s = i * 2 + p
            p1 = 1 - p
            wait(gath_copies(p))

            @pl.when(s + 2 < _NSS)
            def _():
                start(idx_copies(s + 2, p))

            @pl.when(s + 1 < _NSS)
            def _():
                wait(idx_copies(s + 1, p1))
                start(gath_copies(p1))

            def rbody(r4, c):
                compute_row(s, p, r4)
                return c
            lax.fori_loop(0, _G, rbody, 0)
        return carry
    lax.fori_loop(0, _NSS // 2, body, 0)

    pltpu.sync_copy(out_v, out_hbm.at[pl.ds(row0, _RPW), :])


def kernel(target_item, target_cate, target_time, hist_item, hist_cate,
           hist_time, mask, item_table, cate_table, time_table):
    mesh = plsc.VectorSubcoreMesh(core_axis_name="c", subcore_axis_name="s")
    f32 = jnp.float32
    i32 = jnp.int32
    scratch = [
        [[pltpu.VMEM((_SSL,), i32) for _ in range(3)] for _ in range(2)],
        [pltpu.VMEM((_SSL,), f32) for _ in range(2)],
        [[pltpu.VMEM((_SSL, D), f32) for _ in range(3)] for _ in range(2)],
        [pltpu.VMEM((_RPW, D), f32) for _ in range(3)],
        pltpu.VMEM((_RPW,), i32),
        pltpu.VMEM((LP,), f32),
        pltpu.VMEM((_RPW, 6 * D), f32),
        pltpu.SemaphoreType.DMA,
        [pltpu.SemaphoreType.DMA for _ in range(2)],
        [pltpu.SemaphoreType.DMA for _ in range(2)],
    ]
    fn = pl.kernel(_fused_body, mesh=mesh,
                   out_type=jax.ShapeDtypeStruct((B, 6 * D), f32),
                   scratch_types=scratch,
                   compiler_params=pltpu.CompilerParams(
                       use_tc_tiling_on_sc=False,
                       needs_layout_passes=False))
    return fn(hist_item.reshape(-1), hist_cate.reshape(-1),
              hist_time.reshape(-1), target_item, target_cate, target_time,
              mask.reshape(-1), item_table, cate_table, time_table)


# speedup vs baseline: 10.1519x; 1.2057x over previous
import math

import jax
import jax.numpy as jnp
from jax import lax
from jax.experimental import pallas as pl
from jax.experimental.pallas import tpu as pltpu
from jax.experimental.pallas import tpu_sc as plsc

B = 4096
L = 200
D = 16

_NC = 2
_NS = 16
_NW = _NC * _NS
_RPW = B // _NW
_G = 4
_NSS = _RPW // _G
_NCH = 13
_ROWS = _G * L + 8
_GCH = [(0, 128), (128, L - 128)]

_SCALE = 1.0 / math.sqrt(3 * D)
_NEG = -1e9


def _fused_body(hist_item, hist_cate, hist_time, t_item, t_cate, t_time,
                mask2d, item_tb, cate_tb, time_tb, out_hbm,
                idx_bufs, mask_bufs, rows_bufs, t_rows, tidx_v,
                attn_v, out_v, sem_t, sems_idx, sems_g):
    wid = lax.axis_index("s") * _NC + lax.axis_index("c")
    row0 = wid * _RPW
    hists = (hist_item, hist_cate, hist_time)
    tables = (item_tb, cate_tb, time_tb)

    zrow = jnp.zeros((16,), jnp.float32)
    for p in range(2):
        for t in range(3):
            for k in range(8):
                rows_bufs[p][t][_G * L + k] = zrow

    tgts = (t_item, t_cate, t_time)
    for t in range(3):
        pltpu.sync_copy(tgts[t].at[pl.ds(row0, _RPW)], tidx_v)
        pltpu.async_copy(tables[t].at[tidx_v], t_rows[t], sem_t).wait()

    def idx_copies(s, p):
        brow = row0 + s * _G
        cps = [pltpu.make_async_copy(
            hists[t].at[pl.ds(brow, _G), :], idx_bufs[p][t], sems_idx[p])
            for t in range(3)]
        cps.append(pltpu.make_async_copy(
            mask2d.at[pl.ds(brow, _G), :],
            mask_bufs[p].at[:, pl.ds(0, L)], sems_idx[p]))
        return cps

    def gath_copies(p):
        cps = []
        for t in range(3):
            for r in range(_G):
                for (o, n) in _GCH:
                    cps.append(pltpu.make_async_copy(
                        tables[t].at[idx_bufs[p][t].at[r, pl.ds(o, n)]],
                        rows_bufs[p][t].at[pl.ds(r * L + o, n)], sems_g[p]))
        return cps

    def start(cps):
        for c in cps:
            c.start()

    def wait(cps):
        for c in cps:
            c.wait()

    start(idx_copies(0, 0))
    start(idx_copies(1, 1))
    wait(idx_copies(0, 0))
    start(gath_copies(0))

    lanes = lax.iota(jnp.int32, 16)

    def compute_row(s, p, r4):
        base = r4 * L
        rr = s * _G + r4
        tiu = t_rows[0][rr]
        tcu = t_rows[1][rr]
        ttu = t_rows[2][rr]
        ti = tiu * _SCALE
        tc = tcu * _SCALE
        tt = ttu * _SCALE
        ri_ref = rows_bufs[p][0]
        rc_ref = rows_bufs[p][1]
        rt_ref = rows_bufs[p][2]

        def sloop(c, carry):
            lb = base + c * 16
            vec = jnp.zeros((16,), jnp.float32)
            for j in range(16):
                sj = jnp.sum(ri_ref[lb + j] * ti + rc_ref[lb + j] * tc
                             + rt_ref[lb + j] * tt)
                vec = jnp.where(lanes == j, sj, vec)
            attn_v[pl.ds(c * 16, 16)] = vec
            return carry
        lax.fori_loop(0, _NCH, sloop, 0)

        def masked(c):
            sc = attn_v[pl.ds(c * 16, 16)]
            mv = mask_bufs[p][r4, pl.ds(c * 16, 16)]
            valid = mv > 0
            if c == _NCH - 1:
                valid = valid & (lanes < L - (_NCH - 1) * 16)
            return jnp.where(valid, sc, _NEG)

        macc = jnp.full((16,), _NEG, jnp.float32)
        for c in range(_NCH):
            macc = jnp.maximum(macc, masked(c))
        m = jnp.max(macc)

        sacc = jnp.zeros((16,), jnp.float32)
        for c in range(_NCH):
            ec = jnp.exp(masked(c) - m)
            attn_v[pl.ds(c * 16, 16)] = ec
            sacc = sacc + ec
        inv = jnp.ones((16,), jnp.float32) / jnp.sum(sacc)

        def ploop(c, carry):
            ai, ac, at = carry
            lb = c * 16
            av = attn_v[pl.ds(lb, 16)]
            for j in range(16):
                a = av[j]
                ai = ai + a * ri_ref[base + lb + j]
                ac = ac + a * rc_ref[base + lb + j]
                at = at + a * rt_ref[base + lb + j]
            return (ai, ac, at)
        z = jnp.zeros((16,), jnp.float32)
        ai, ac, at = lax.fori_loop(0, _NCH, ploop, (z, z, z))

        out_v[rr, pl.ds(0, 16)] = tiu
        out_v[rr, pl.ds(16, 16)] = tcu
        out_v[rr, pl.ds(32, 16)] = ttu
        out_v[rr, pl.ds(48, 16)] = ai * inv
        out_v[rr, pl.ds(64, 16)] = ac * inv
        out_v[rr, pl.ds(80, 16)] = at * inv

    def body(i, carry):
        for p in range(2):
            s = i * 2 + p
            p1 = 1 - p
            wait(gath_copies(p))

            @pl.when(s + 2 < _NSS)
            def _():
                start(idx_copies(s + 2, p))

            @pl.when(s + 1 < _NSS)
            def _():
                wait(idx_copies(s + 1, p1))
                start(gath_copies(p1))

            def rbody(r4, c):
                compute_row(s, p, r4)
                return c
            lax.fori_loop(0, _G, rbody, 0)
        return carry
    lax.fori_loop(0, _NSS // 2, body, 0)

    pltpu.sync_copy(out_v, out_hbm.at[pl.ds(row0, _RPW), :])


def kernel(target_item, target_cate, target_time, hist_item, hist_cate,
           hist_time, mask, item_table, cate_table, time_table):
    mesh = plsc.VectorSubcoreMesh(core_axis_name="c", subcore_axis_name="s")
    f32 = jnp.float32
    i32 = jnp.int32
    scratch = [
        [[pltpu.VMEM((_G, L), i32) for _ in range(3)] for _ in range(2)],
        [pltpu.VMEM((_G, _NCH * 16), f32) for _ in range(2)],
        [[pltpu.VMEM((_ROWS, D), f32) for _ in range(3)] for _ in range(2)],
        [pltpu.VMEM((_RPW, D), f32) for _ in range(3)],
        pltpu.VMEM((_RPW,), i32),
        pltpu.VMEM((_NCH * 16,), f32),
        pltpu.VMEM((_RPW, 6 * D), f32),
        pltpu.SemaphoreType.DMA,
        [pltpu.SemaphoreType.DMA for _ in range(2)],
        [pltpu.SemaphoreType.DMA for _ in range(2)],
    ]
    fn = pl.kernel(_fused_body, mesh=mesh,
                   out_type=jax.ShapeDtypeStruct((B, 6 * D), f32),
                   scratch_types=scratch,
                   compiler_params=pltpu.CompilerParams(
                       use_tc_tiling_on_sc=False,
                       needs_layout_passes=False))
    return fn(hist_item, hist_cate, hist_time,
              target_item, target_cate, target_time,
              mask, item_table, cate_table, time_table)
